# baseline (speedup 1.0000x reference)
import numpy as np

import jax
import jax.numpy as jnp
from jax import lax
from jax.experimental import pallas as pl
from jax.experimental.pallas import tpu as pltpu
from jax.experimental.pallas import tpu_sc as plsc

B, S, D = 4, 4096, 1024
R = 2048
OUT_ROWS = B * (R + 1)
NT = D // 128

NC, NS = 2, 16
NW = NC * NS
RPW = (B * R) // NW
G = 16
NCHUNK = RPW // G
NBUF = 2

_AUX = np.zeros(72, np.int32)
_AUX[np.arange(B) * 8] = np.arange(B, dtype=np.int32) * S
_AUX[40 + np.arange(B * NT)] = (
    np.arange(NT, dtype=np.int32)[None, :] * B
    + np.arange(B, dtype=np.int32)[:, None]).reshape(-1)


def _body(val_hbm, pos_hbm, ridx_hbm, aux_hbm, out_hbm,
          ridx_v, pidx_v, vidx_v, oidx_v, gvidx_v, gzidx_v, goidx_v,
          grows_v, gprows_v, vrows, prows, srows,
          sem_v0, sem_v1, sem_p0, sem_p1, sem_o0, sem_o1, sem_g):
  wid = lax.axis_index("s") * NC + lax.axis_index("c")
  b = wid // (NW // B)
  t0 = wid * RPW
  lane = lax.iota(jnp.int32, 16)

  @pl.when(lax.rem(wid, NW // B) == lax.rem(b, 2))
  def _():
    pltpu.sync_copy(aux_hbm.at[pl.ds(pl.multiple_of(8 * b, 8), 1)], gvidx_v)
    pltpu.sync_copy(aux_hbm.at[pl.ds(32, 1)], gzidx_v)
    pltpu.sync_copy(aux_hbm.at[pl.ds(pl.multiple_of(40 + 8 * b, 8), NT)],
                    goidx_v)
    cp_v = pltpu.async_copy(val_hbm.at[gvidx_v], grows_v, sem_g)
    cp_p = pltpu.async_copy(pos_hbm.at[gzidx_v], gprows_v, sem_g)
    cp_v.wait()
    cp_p.wait()

    @plsc.parallel_loop(0, D // 16, 1, unroll=4)
    def _(u):
      o = pl.multiple_of(u << 4, 16)
      oo = pl.multiple_of((u & 7) << 4, 16)
      gsrow_v = grows_v[0, pl.ds(o, 16)] + gprows_v[0, pl.ds(o, 16)]
      srows[NBUF, (u >> 3) & (NT - 1), pl.ds(oo, 16)] = gsrow_v

    cp_o = pltpu.async_copy(
        srows.at[NBUF].at[pl.ds(0, NT)], out_hbm.at[goidx_v], sem_g)
    cp_o.wait()

  tq = pl.multiple_of((wid % (NW // B)) * 2 * B * 128 + b * 128, 128)
  pltpu.sync_copy(ridx_hbm.at[pl.ds(tq, 128)], ridx_v.at[pl.ds(0, 128)])
  pltpu.sync_copy(ridx_hbm.at[pl.ds(tq + B * 128, 128)],
                  ridx_v.at[pl.ds(128, 128)])

  base_p1 = t0 - b * R + 1
  for k in range(NCHUNK):
    pr = ridx_v[pl.ds(k * G, 16)] + 1
    pidx_v[k, pl.ds(0, 16)] = pr
    vidx_v[k, pl.ds(0, 16)] = pr + b * S
    for j in range(G * NT // 16):
      s = lane + j * 16
      oidx_v[k, pl.ds(j * 16, 16)] = (
          ((base_p1 + k * G + (s >> 3)) << 5) + ((s & 7) << 2) + b)

  sem_v = (sem_v0, sem_v1)
  sem_p = (sem_p0, sem_p1)
  sem_o = (sem_o0, sem_o1)

  def fire_gather(c, ib):
    return (pltpu.async_copy(val_hbm.at[vidx_v.at[c]], vrows.at[ib], sem_v[ib]),
            pltpu.async_copy(pos_hbm.at[pidx_v.at[c]], prows.at[ib], sem_p[ib]))

  pending_scatter = [None] * NBUF
  gathers = [None] * NBUF
  gathers[0] = fire_gather(0, 0)
  for c in range(NCHUNK):
    ib = c % NBUF
    nc = c + NBUF - 1
    if nc < NCHUNK:
      nb = nc % NBUF
      if pending_scatter[nb] is not None:
        pending_scatter[nb].wait()
        pending_scatter[nb] = None
      gathers[nb] = fire_gather(nc, nb)
    gathers[ib][0].wait()
    gathers[ib][1].wait()

    @plsc.parallel_loop(0, G * (D // 16), 1, unroll=16)
    def _(u):
      i = u >> 6
      o = pl.multiple_of((u & (D // 16 - 1)) << 4, 16)
      oo = pl.multiple_of((u & 7) << 4, 16)
      srow_v = vrows[ib, i, pl.ds(o, 16)] + prows[ib, i, pl.ds(o, 16)]
      srows[ib, (i << 3) + ((u >> 3) & (NT - 1)), pl.ds(oo, 16)] = srow_v

    pending_scatter[ib] = pltpu.async_copy(
        srows.at[ib], out_hbm.at[oidx_v.at[c]], sem_o[ib])
  for ps in pending_scatter:
    if ps is not None:
      ps.wait()


@jax.jit
def _run(val2, pos_emb, ridx, aux):
  mesh = plsc.VectorSubcoreMesh(core_axis_name="c", subcore_axis_name="s")
  f = pl.kernel(
      _body,
      out_type=jax.ShapeDtypeStruct((OUT_ROWS * NT, 128), jnp.float32),
      mesh=mesh,
      scratch_types=[
          pltpu.VMEM((RPW,), jnp.int32),
          pltpu.VMEM((NCHUNK, G), jnp.int32),
          pltpu.VMEM((NCHUNK, G), jnp.int32),
          pltpu.VMEM((NCHUNK, G * NT), jnp.int32),
          pltpu.VMEM((1,), jnp.int32),
          pltpu.VMEM((1,), jnp.int32),
          pltpu.VMEM((NT,), jnp.int32),
          pltpu.VMEM((1, D), jnp.float32),
          pltpu.VMEM((1, D), jnp.float32),
          pltpu.VMEM((NBUF, G, D), jnp.float32),
          pltpu.VMEM((NBUF, G, D), jnp.float32),
          pltpu.VMEM((NBUF + 1, G * NT, 128), jnp.float32),
          pltpu.SemaphoreType.DMA,
          pltpu.SemaphoreType.DMA,
          pltpu.SemaphoreType.DMA,
          pltpu.SemaphoreType.DMA,
          pltpu.SemaphoreType.DMA,
          pltpu.SemaphoreType.DMA,
          pltpu.SemaphoreType.DMA,
      ],
  )
  return f(val2, pos_emb, ridx, aux)


def kernel(val, pos_emb, remain_idx):
  val2 = val.reshape(B * S, D)
  ridx = (remain_idx.astype(jnp.int32)
          .reshape(B, R // 128, 128)
          .transpose(1, 0, 2)
          .reshape(B * R))
  out128 = _run(val2, pos_emb, ridx, jnp.asarray(_AUX))
  return (out128.reshape(R + 1, NT, B, 128)
          .transpose(2, 0, 1, 3)
          .reshape(B, R + 1, D))


# speedup vs baseline: 1.0452x; 1.0452x over previous
import numpy as np

import jax
import jax.numpy as jnp
from jax import lax
from jax.experimental import pallas as pl
from jax.experimental.pallas import tpu as pltpu
from jax.experimental.pallas import tpu_sc as plsc

B, S, D = 4, 4096, 1024
R = 2048
OUT_ROWS = B * (R + 1)
NT = D // 128

NC, NS = 2, 16
NW = NC * NS
RPW = (B * R) // NW
G = 16
NCHUNK = RPW // G
NBUF = 2

_AUX = np.zeros(72, np.int32)
_AUX[np.arange(B) * 8] = np.arange(B, dtype=np.int32) * S
_AUX[40 + np.arange(B * NT)] = (
    np.arange(NT, dtype=np.int32)[None, :] * B
    + np.arange(B, dtype=np.int32)[:, None]).reshape(-1)


def _body(val_hbm, pos_hbm, ridx_hbm, aux_hbm, out_hbm,
          ridx_v, pidx_v, vidx_v, oidx_v, gvidx_v, gzidx_v, goidx_v,
          grows_v, gprows_v, vrows, prows, srows,
          sem_v0, sem_v1, sem_p0, sem_p1, sem_o0, sem_o1, sem_g):
  wid = lax.axis_index("s") * NC + lax.axis_index("c")
  b = wid // (NW // B)
  t0 = wid * RPW
  lane = lax.iota(jnp.int32, 16)

  is_g = lax.rem(wid, NW // B) == lax.rem(b, 2)

  @pl.when(is_g)
  def _():
    pltpu.async_copy(aux_hbm.at[pl.ds(pl.multiple_of(8 * b, 8), 1)],
                     gvidx_v, sem_g)
    pltpu.async_copy(aux_hbm.at[pl.ds(32, 1)], gzidx_v, sem_g)
    pltpu.async_copy(aux_hbm.at[pl.ds(pl.multiple_of(40 + 8 * b, 8), NT)],
                     goidx_v, sem_g)

  tq = pl.multiple_of((wid % (NW // B)) * 2 * B * 128 + b * 128, 128)
  pltpu.sync_copy(ridx_hbm.at[pl.ds(tq, 128)], ridx_v.at[pl.ds(0, 128)])
  pltpu.sync_copy(ridx_hbm.at[pl.ds(tq + B * 128, 128)],
                  ridx_v.at[pl.ds(128, 128)])

  @pl.when(is_g)
  def _():
    pltpu.make_async_copy(aux_hbm.at[pl.ds(0, 1)], gvidx_v, sem_g).wait()
    pltpu.make_async_copy(aux_hbm.at[pl.ds(0, 1)], gzidx_v, sem_g).wait()
    pltpu.make_async_copy(aux_hbm.at[pl.ds(0, NT)], goidx_v, sem_g).wait()
    pltpu.async_copy(val_hbm.at[gvidx_v], grows_v, sem_g)
    pltpu.async_copy(pos_hbm.at[gzidx_v], gprows_v, sem_g)

  base_p1 = t0 - b * R + 1
  for k in range(NCHUNK):
    pr = ridx_v[pl.ds(k * G, 16)] + 1
    pidx_v[k, pl.ds(0, 16)] = pr
    vidx_v[k, pl.ds(0, 16)] = pr + b * S
    for j in range(G * NT // 16):
      s = lane + j * 16
      oidx_v[k, pl.ds(j * 16, 16)] = (
          ((base_p1 + k * G + (s >> 3)) << 5) + ((s & 7) << 2) + b)

  sem_v = (sem_v0, sem_v1)
  sem_p = (sem_p0, sem_p1)
  sem_o = (sem_o0, sem_o1)

  def fire_gather(c, ib):
    return (pltpu.async_copy(val_hbm.at[vidx_v.at[c]], vrows.at[ib], sem_v[ib]),
            pltpu.async_copy(pos_hbm.at[pidx_v.at[c]], prows.at[ib], sem_p[ib]))

  pending_scatter = [None] * NBUF
  gathers = [None] * NBUF
  gathers[0] = fire_gather(0, 0)
  for c in range(NCHUNK):
    ib = c % NBUF
    nc = c + NBUF - 1
    if nc < NCHUNK:
      nb = nc % NBUF
      if pending_scatter[nb] is not None:
        pending_scatter[nb].wait()
        pending_scatter[nb] = None
      gathers[nb] = fire_gather(nc, nb)
    gathers[ib][0].wait()
    gathers[ib][1].wait()

    @plsc.parallel_loop(0, G * (D // 16), 1, unroll=8)
    def _(u):
      i = u >> 6
      o = pl.multiple_of((u & (D // 16 - 1)) << 4, 16)
      oo = pl.multiple_of((u & 7) << 4, 16)
      srow_v = vrows[ib, i, pl.ds(o, 16)] + prows[ib, i, pl.ds(o, 16)]
      srows[ib, (i << 3) + ((u >> 3) & (NT - 1)), pl.ds(oo, 16)] = srow_v

    pending_scatter[ib] = pltpu.async_copy(
        srows.at[ib], out_hbm.at[oidx_v.at[c]], sem_o[ib])
  for ps in pending_scatter:
    if ps is not None:
      ps.wait()

  @pl.when(is_g)
  def _():
    pltpu.make_async_copy(val_hbm.at[gvidx_v], grows_v, sem_g).wait()
    pltpu.make_async_copy(pos_hbm.at[gzidx_v], gprows_v, sem_g).wait()

    @plsc.parallel_loop(0, D // 16, 1, unroll=4)
    def _(u):
      o = pl.multiple_of(u << 4, 16)
      oo = pl.multiple_of((u & 7) << 4, 16)
      gsrow_v = grows_v[0, pl.ds(o, 16)] + gprows_v[0, pl.ds(o, 16)]
      srows[NBUF, (u >> 3) & (NT - 1), pl.ds(oo, 16)] = gsrow_v

    cp_o = pltpu.async_copy(
        srows.at[NBUF].at[pl.ds(0, NT)], out_hbm.at[goidx_v], sem_g)
    cp_o.wait()


@jax.jit
def _run(val2, pos_emb, ridx, aux):
  mesh = plsc.VectorSubcoreMesh(core_axis_name="c", subcore_axis_name="s")
  f = pl.kernel(
      _body,
      out_type=jax.ShapeDtypeStruct((OUT_ROWS * NT, 128), jnp.float32),
      mesh=mesh,
      scratch_types=[
          pltpu.VMEM((RPW,), jnp.int32),
          pltpu.VMEM((NCHUNK, G), jnp.int32),
          pltpu.VMEM((NCHUNK, G), jnp.int32),
          pltpu.VMEM((NCHUNK, G * NT), jnp.int32),
          pltpu.VMEM((1,), jnp.int32),
          pltpu.VMEM((1,), jnp.int32),
          pltpu.VMEM((NT,), jnp.int32),
          pltpu.VMEM((1, D), jnp.float32),
          pltpu.VMEM((1, D), jnp.float32),
          pltpu.VMEM((NBUF, G, D), jnp.float32),
          pltpu.VMEM((NBUF, G, D), jnp.float32),
          pltpu.VMEM((NBUF + 1, G * NT, 128), jnp.float32),
          pltpu.SemaphoreType.DMA,
          pltpu.SemaphoreType.DMA,
          pltpu.SemaphoreType.DMA,
          pltpu.SemaphoreType.DMA,
          pltpu.SemaphoreType.DMA,
          pltpu.SemaphoreType.DMA,
          pltpu.SemaphoreType.DMA,
      ],
  )
  return f(val2, pos_emb, ridx, aux)


def kernel(val, pos_emb, remain_idx):
  val2 = val.reshape(B * S, D)
  ridx = (remain_idx.astype(jnp.int32)
          .reshape(B, R // 128, 128)
          .transpose(1, 0, 2)
          .reshape(B * R))
  out128 = _run(val2, pos_emb, ridx, jnp.asarray(_AUX))
  return (out128.reshape(R + 1, NT, B, 128)
          .transpose(2, 0, 1, 3)
          .reshape(B, R + 1, D))


# speedup vs baseline: 1.0576x; 1.0118x over previous
import numpy as np

import jax
import jax.numpy as jnp
from jax import lax
from jax.experimental import pallas as pl
from jax.experimental.pallas import tpu as pltpu
from jax.experimental.pallas import tpu_sc as plsc

B, S, D = 4, 4096, 1024
R = 2048
OUT_ROWS = B * (R + 1)
NT = D // 128

NC, NS = 2, 16
NW = NC * NS
RPW = (B * R) // NW
G = 16
NCHUNK = RPW // G
NBUF = 2

_AUX = np.zeros(72, np.int32)
_AUX[np.arange(B) * 8] = np.arange(B, dtype=np.int32) * S
_AUX[40 + np.arange(B * NT)] = (
    np.arange(NT, dtype=np.int32)[None, :] * B
    + np.arange(B, dtype=np.int32)[:, None]).reshape(-1)


def _body(val_hbm, pos_hbm, ridx_hbm, aux_hbm, out_hbm,
          ridx_v, pidx_v, vidx_v, oidx_v, gvidx_v, gzidx_v, goidx_v,
          grows_v, gprows_v, vrows, prows, srows,
          sem_v0, sem_v1, sem_p0, sem_p1, sem_o0, sem_o1, sem_g, sem_r):
  wid = lax.axis_index("s") * NC + lax.axis_index("c")
  b = wid // (NW // B)
  t0 = wid * RPW
  lane = lax.iota(jnp.int32, 16)

  is_g = lax.rem(wid, NW // B) == lax.rem(b, 2)

  @pl.when(is_g)
  def _():
    pltpu.async_copy(aux_hbm.at[pl.ds(pl.multiple_of(8 * b, 8), 1)],
                     gvidx_v, sem_g)
    pltpu.async_copy(aux_hbm.at[pl.ds(32, 1)], gzidx_v, sem_g)
    pltpu.async_copy(aux_hbm.at[pl.ds(pl.multiple_of(40 + 8 * b, 8), NT)],
                     goidx_v, sem_g)

  tq = pl.multiple_of((wid % (NW // B)) * 2 * B * 128 + b * 128, 128)
  cp_r0 = pltpu.async_copy(ridx_hbm.at[pl.ds(tq, 128)],
                           ridx_v.at[pl.ds(0, 128)], sem_r)
  cp_r1 = pltpu.async_copy(ridx_hbm.at[pl.ds(tq + B * 128, 128)],
                           ridx_v.at[pl.ds(128, 128)], sem_r)

  @pl.when(is_g)
  def _():
    pltpu.make_async_copy(aux_hbm.at[pl.ds(0, 1)], gvidx_v, sem_g).wait()
    pltpu.make_async_copy(aux_hbm.at[pl.ds(0, 1)], gzidx_v, sem_g).wait()
    pltpu.make_async_copy(aux_hbm.at[pl.ds(0, NT)], goidx_v, sem_g).wait()
    pltpu.async_copy(val_hbm.at[gvidx_v], grows_v, sem_g)
    pltpu.async_copy(pos_hbm.at[gzidx_v], gprows_v, sem_g)

  base_p1 = t0 - b * R + 1
  cp_r0.wait()
  cp_r1.wait()

  def gather_idx(k):
    pr = ridx_v[pl.ds(k * G, 16)] + 1
    pidx_v[k, pl.ds(0, 16)] = pr
    vidx_v[k, pl.ds(0, 16)] = pr + b * S

  def scatter_idx(k):
    for j in range(G * NT // 16):
      s = lane + j * 16
      oidx_v[k, pl.ds(j * 16, 16)] = (
          ((base_p1 + k * G + (s >> 3)) << 5) + ((s & 7) << 2) + b)

  sem_v = (sem_v0, sem_v1)
  sem_p = (sem_p0, sem_p1)
  sem_o = (sem_o0, sem_o1)

  def fire_gather(c, ib):
    return (pltpu.async_copy(val_hbm.at[vidx_v.at[c]], vrows.at[ib], sem_v[ib]),
            pltpu.async_copy(pos_hbm.at[pidx_v.at[c]], prows.at[ib], sem_p[ib]))

  pending_scatter = [None] * NBUF
  gathers = [None] * NBUF
  gather_idx(0)
  gathers[0] = fire_gather(0, 0)
  for k in range(1, NCHUNK):
    gather_idx(k)
  for k in range(NCHUNK):
    scatter_idx(k)
  for c in range(NCHUNK):
    ib = c % NBUF
    nc = c + NBUF - 1
    if nc < NCHUNK:
      nb = nc % NBUF
      if pending_scatter[nb] is not None:
        pending_scatter[nb].wait()
        pending_scatter[nb] = None
      gathers[nb] = fire_gather(nc, nb)
    gathers[ib][0].wait()
    gathers[ib][1].wait()

    @plsc.parallel_loop(0, G * (D // 16), 1, unroll=8)
    def _(u):
      i = u >> 6
      o = pl.multiple_of((u & (D // 16 - 1)) << 4, 16)
      oo = pl.multiple_of((u & 7) << 4, 16)
      srow_v = vrows[ib, i, pl.ds(o, 16)] + prows[ib, i, pl.ds(o, 16)]
      srows[ib, (i << 3) + ((u >> 3) & (NT - 1)), pl.ds(oo, 16)] = srow_v

    pending_scatter[ib] = pltpu.async_copy(
        srows.at[ib], out_hbm.at[oidx_v.at[c]], sem_o[ib])
  for ps in pending_scatter:
    if ps is not None:
      ps.wait()

  @pl.when(is_g)
  def _():
    pltpu.make_async_copy(val_hbm.at[gvidx_v], grows_v, sem_g).wait()
    pltpu.make_async_copy(pos_hbm.at[gzidx_v], gprows_v, sem_g).wait()

    @plsc.parallel_loop(0, D // 16, 1, unroll=4)
    def _(u):
      o = pl.multiple_of(u << 4, 16)
      oo = pl.multiple_of((u & 7) << 4, 16)
      gsrow_v = grows_v[0, pl.ds(o, 16)] + gprows_v[0, pl.ds(o, 16)]
      srows[NBUF, (u >> 3) & (NT - 1), pl.ds(oo, 16)] = gsrow_v

    cp_o = pltpu.async_copy(
        srows.at[NBUF].at[pl.ds(0, NT)], out_hbm.at[goidx_v], sem_g)
    cp_o.wait()


@jax.jit
def _run(val2, pos_emb, ridx, aux):
  mesh = plsc.VectorSubcoreMesh(core_axis_name="c", subcore_axis_name="s")
  f = pl.kernel(
      _body,
      out_type=jax.ShapeDtypeStruct((OUT_ROWS * NT, 128), jnp.float32),
      mesh=mesh,
      scratch_types=[
          pltpu.VMEM((RPW,), jnp.int32),
          pltpu.VMEM((NCHUNK, G), jnp.int32),
          pltpu.VMEM((NCHUNK, G), jnp.int32),
          pltpu.VMEM((NCHUNK, G * NT), jnp.int32),
          pltpu.VMEM((1,), jnp.int32),
          pltpu.VMEM((1,), jnp.int32),
          pltpu.VMEM((NT,), jnp.int32),
          pltpu.VMEM((1, D), jnp.float32),
          pltpu.VMEM((1, D), jnp.float32),
          pltpu.VMEM((NBUF, G, D), jnp.float32),
          pltpu.VMEM((NBUF, G, D), jnp.float32),
          pltpu.VMEM((NBUF + 1, G * NT, 128), jnp.float32),
          pltpu.SemaphoreType.DMA,
          pltpu.SemaphoreType.DMA,
          pltpu.SemaphoreType.DMA,
          pltpu.SemaphoreType.DMA,
          pltpu.SemaphoreType.DMA,
          pltpu.SemaphoreType.DMA,
          pltpu.SemaphoreType.DMA,
          pltpu.SemaphoreType.DMA,
      ],
  )
  return f(val2, pos_emb, ridx, aux)


def kernel(val, pos_emb, remain_idx):
  val2 = val.reshape(B * S, D)
  ridx = (remain_idx.astype(jnp.int32)
          .reshape(B, R // 128, 128)
          .transpose(1, 0, 2)
          .reshape(B * R))
  out128 = _run(val2, pos_emb, ridx, jnp.asarray(_AUX))
  return (out128.reshape(R + 1, NT, B, 128)
          .transpose(2, 0, 1, 3)
          .reshape(B, R + 1, D))


# speedup vs baseline: 1.1202x; 1.0592x over previous
import numpy as np

import jax
import jax.numpy as jnp
from jax import lax
from jax.experimental import pallas as pl
from jax.experimental.pallas import tpu as pltpu
from jax.experimental.pallas import tpu_sc as plsc

B, S, D = 4, 4096, 1024
R = 2048
OUT_ROWS = B * (R + 1)
NT = D // 128

NC, NS = 2, 16
NW = NC * NS
RPW = (B * R) // NW
G = 16
NCHUNK = RPW // G
NBUF = 2

_AUX = np.zeros(72, np.int32)
_AUX[np.arange(B) * 8] = np.arange(B, dtype=np.int32) * S
_AUX[40 + np.arange(B * NT)] = (
    np.arange(NT, dtype=np.int32)[None, :] * B
    + np.arange(B, dtype=np.int32)[:, None]).reshape(-1)


def _body(val_hbm, pos_hbm, ridx_hbm, aux_hbm, out_hbm,
          ridx_v, pidx_v, vidx_v, oidx_v, gvidx_v, gzidx_v, goidx_v,
          grows_v, gprows_v, vrows, prows, srows,
          sem_v0, sem_v1, sem_p0, sem_p1, sem_o0, sem_o1, sem_g, sem_r):
  wid = lax.axis_index("s") * NC + lax.axis_index("c")
  b = wid // (NW // B)
  t0 = wid * RPW
  lane = lax.iota(jnp.int32, 16)

  is_g = lax.rem(wid, NW // B) == lax.rem(b, 2)

  @pl.when(is_g)
  def _():
    pltpu.async_copy(aux_hbm.at[pl.ds(pl.multiple_of(8 * b, 8), 1)],
                     gvidx_v, sem_g)
    pltpu.async_copy(aux_hbm.at[pl.ds(32, 1)], gzidx_v, sem_g)
    pltpu.async_copy(aux_hbm.at[pl.ds(pl.multiple_of(40 + 8 * b, 8), NT)],
                     goidx_v, sem_g)

  tq = pl.multiple_of((wid % (NW // B)) * 2 * B * 128 + b * 128, 128)
  cp_r0 = pltpu.async_copy(ridx_hbm.at[pl.ds(tq, 128)],
                           ridx_v.at[pl.ds(0, 128)], sem_r)
  cp_r1 = pltpu.async_copy(ridx_hbm.at[pl.ds(tq + B * 128, 128)],
                           ridx_v.at[pl.ds(128, 128)], sem_r)

  @pl.when(is_g)
  def _():
    pltpu.make_async_copy(aux_hbm.at[pl.ds(0, 1)], gvidx_v, sem_g).wait()
    pltpu.make_async_copy(aux_hbm.at[pl.ds(0, 1)], gzidx_v, sem_g).wait()
    pltpu.make_async_copy(aux_hbm.at[pl.ds(0, NT)], goidx_v, sem_g).wait()
    pltpu.async_copy(val_hbm.at[gvidx_v], grows_v, sem_g)
    pltpu.async_copy(pos_hbm.at[gzidx_v], gprows_v, sem_g)

  base_p1 = t0 - b * R + 1
  cp_r0.wait()
  cp_r1.wait()

  @plsc.parallel_loop(0, NCHUNK, 1, unroll=2)
  def _(k):
    pr = ridx_v[pl.ds(pl.multiple_of(k * G, 16), 16)] + 1
    pidx_v[k, pl.ds(0, 16)] = pr
    vidx_v[k, pl.ds(0, 16)] = pr + b * S

  sem_v = (sem_v0, sem_v1)
  sem_p = (sem_p0, sem_p1)
  sem_o = (sem_o0, sem_o1)

  def fire_gather(c, ib):
    pltpu.async_copy(val_hbm.at[vidx_v.at[c]], vrows.at[ib], sem_v[ib])
    pltpu.async_copy(pos_hbm.at[pidx_v.at[c]], prows.at[ib], sem_p[ib])

  def wait_gather(ib):
    pltpu.make_async_copy(
        val_hbm.at[vidx_v.at[0]], vrows.at[ib], sem_v[ib]).wait()
    pltpu.make_async_copy(
        pos_hbm.at[pidx_v.at[0]], prows.at[ib], sem_p[ib]).wait()

  def fire_scatter(c, ib):
    pltpu.async_copy(srows.at[ib], out_hbm.at[oidx_v.at[c]], sem_o[ib])

  def wait_scatter(ib):
    pltpu.make_async_copy(
        srows.at[ib], out_hbm.at[oidx_v.at[0]], sem_o[ib]).wait()

  def add_chunk(ib):
    @plsc.parallel_loop(0, G * (D // 16), 1, unroll=8)
    def _(u):
      i = u >> 6
      o = pl.multiple_of((u & (D // 16 - 1)) << 4, 16)
      oo = pl.multiple_of((u & 7) << 4, 16)
      srow_v = vrows[ib, i, pl.ds(o, 16)] + prows[ib, i, pl.ds(o, 16)]
      srows[ib, (i << 3) + ((u >> 3) & (NT - 1)), pl.ds(oo, 16)] = srow_v

  fire_gather(0, 0)

  @plsc.parallel_loop(0, NCHUNK * (G * NT // 16), 1, unroll=2)
  def _(w):
    k = w >> 3
    s = lane + ((w & 7) << 4)
    oidx_v[k, pl.ds(pl.multiple_of((w & 7) << 4, 16), 16)] = (
        ((base_p1 + k * G + (s >> 3)) << 5) + ((s & 7) << 2) + b)

  fire_gather(1, 1)
  wait_gather(0)
  add_chunk(0)
  fire_scatter(0, 0)

  fire_gather(2, 0)
  wait_gather(1)
  add_chunk(1)
  fire_scatter(1, 1)

  @pl.loop(1, (NCHUNK - 2) // 2)
  def _(cc):
    ca = 2 * cc
    fire_gather(ca + 1, 1)
    wait_gather(0)
    wait_scatter(0)
    add_chunk(0)
    fire_scatter(ca, 0)
    fire_gather(ca + 2, 0)
    wait_gather(1)
    wait_scatter(1)
    add_chunk(1)
    fire_scatter(ca + 1, 1)

  fire_gather(NCHUNK - 1, 1)
  wait_gather(0)
  wait_scatter(0)
  add_chunk(0)
  fire_scatter(NCHUNK - 2, 0)

  wait_gather(1)
  wait_scatter(1)
  add_chunk(1)
  fire_scatter(NCHUNK - 1, 1)

  wait_scatter(0)
  wait_scatter(1)

  @pl.when(is_g)
  def _():
    pltpu.make_async_copy(val_hbm.at[gvidx_v], grows_v, sem_g).wait()
    pltpu.make_async_copy(pos_hbm.at[gzidx_v], gprows_v, sem_g).wait()

    @plsc.parallel_loop(0, D // 16, 1, unroll=4)
    def _(u):
      o = pl.multiple_of(u << 4, 16)
      oo = pl.multiple_of((u & 7) << 4, 16)
      gsrow_v = grows_v[0, pl.ds(o, 16)] + gprows_v[0, pl.ds(o, 16)]
      srows[NBUF, (u >> 3) & (NT - 1), pl.ds(oo, 16)] = gsrow_v

    cp_o = pltpu.async_copy(
        srows.at[NBUF].at[pl.ds(0, NT)], out_hbm.at[goidx_v], sem_g)
    cp_o.wait()


@jax.jit
def _run(val2, pos_emb, ridx, aux):
  mesh = plsc.VectorSubcoreMesh(core_axis_name="c", subcore_axis_name="s")
  f = pl.kernel(
      _body,
      out_type=jax.ShapeDtypeStruct((OUT_ROWS * NT, 128), jnp.float32),
      mesh=mesh,
      scratch_types=[
          pltpu.VMEM((RPW,), jnp.int32),
          pltpu.VMEM((NCHUNK, G), jnp.int32),
          pltpu.VMEM((NCHUNK, G), jnp.int32),
          pltpu.VMEM((NCHUNK, G * NT), jnp.int32),
          pltpu.VMEM((1,), jnp.int32),
          pltpu.VMEM((1,), jnp.int32),
          pltpu.VMEM((NT,), jnp.int32),
          pltpu.VMEM((1, D), jnp.float32),
          pltpu.VMEM((1, D), jnp.float32),
          pltpu.VMEM((NBUF, G, D), jnp.float32),
          pltpu.VMEM((NBUF, G, D), jnp.float32),
          pltpu.VMEM((NBUF + 1, G * NT, 128), jnp.float32),
          pltpu.SemaphoreType.DMA,
          pltpu.SemaphoreType.DMA,
          pltpu.SemaphoreType.DMA,
          pltpu.SemaphoreType.DMA,
          pltpu.SemaphoreType.DMA,
          pltpu.SemaphoreType.DMA,
          pltpu.SemaphoreType.DMA,
          pltpu.SemaphoreType.DMA,
      ],
  )
  return f(val2, pos_emb, ridx, aux)


def kernel(val, pos_emb, remain_idx):
  val2 = val.reshape(B * S, D)
  ridx = (remain_idx.astype(jnp.int32)
          .reshape(B, R // 128, 128)
          .transpose(1, 0, 2)
          .reshape(B * R))
  out128 = _run(val2, pos_emb, ridx, jnp.asarray(_AUX))
  return (out128.reshape(R + 1, NT, B, 128)
          .transpose(2, 0, 1, 3)
          .reshape(B, R + 1, D))
